# Initial kernel scaffold; baseline (speedup 1.0000x reference)
import jax
import jax.numpy as jnp
from jax.experimental import pallas as pl


def kernel(movements, utterances, votes, hive_values, locations):
    raise NotImplementedError("write your pallas kernel here")


# speedup vs baseline: 3.4241x; 3.4241x over previous
import functools

import jax
import jax.numpy as jnp
from jax import lax
from jax.experimental import pallas as pl
from jax.experimental.pallas import tpu as pltpu
from jax.experimental.pallas import tpu_sc as plsc

_B = 512
_A = 64
_H = 16
_E = 80
_L = 16
_NW = 32
_BPW = _B // _NW

_D, _K, _T = 100.0, 30.0, 0.7


def _sc_bee_game(movements, votes, hv):
    mesh = plsc.VectorSubcoreMesh(core_axis_name="c", subcore_axis_name="s")

    @functools.partial(
        pl.kernel,
        out_type=(
            jax.ShapeDtypeStruct((_NW, _L), jnp.float32),
            jax.ShapeDtypeStruct((_B,), jnp.float32),
        ),
        mesh=mesh,
        compiler_params=pltpu.CompilerParams(
            needs_layout_passes=False, use_tc_tiling_on_sc=False),
        scratch_types=[
            pltpu.VMEM((_BPW, _A, _H), jnp.float32),
            pltpu.VMEM((_BPW, _E, 2), jnp.float32),
            pltpu.VMEM((_BPW, _H), jnp.float32),
            pltpu.VMEM((_BPW, _H), jnp.float32),
            pltpu.VMEM((_L,), jnp.float32),
            pltpu.VMEM((_L,), jnp.float32),
        ],
    )
    def k(mov_hbm, votes_hbm, hv_hbm, part_out, mf_out,
          votes_v, mov_v, hv_v, counts_v, mf_st, part_st):
        wid = lax.axis_index("s") * 2 + lax.axis_index("c")
        base = wid * _BPW

        pltpu.sync_copy(votes_hbm.at[pl.ds(base, _BPW)], votes_v)
        pltpu.sync_copy(mov_hbm.at[pl.ds(base, _BPW)], mov_v)
        pltpu.sync_copy(hv_hbm.at[pl.ds(base, _BPW)], hv_v)

        lane = lax.broadcasted_iota(jnp.int32, (_L,), 0)
        fzero = jnp.zeros((_L,), jnp.float32)
        fone = jnp.ones((_L,), jnp.float32)
        izero = jnp.zeros((_L,), jnp.int32)
        ione = jnp.ones((_L,), jnp.int32)

        for r in range(_BPW):
            counts_v[r, :] = fzero

        def agent_body(a, values_acc):
            av = jnp.full((_L,), a, jnp.int32)
            bv = jnp.full((_L,), -jnp.inf, jnp.float32)
            bi = izero
            for h in range(_H):
                v = plsc.load_gather(
                    votes_v, [lane, av, jnp.full((_L,), h, jnp.int32)])
                better = v > bv
                bv = jnp.where(better, v, bv)
                bi = jnp.where(better, jnp.full((_L,), h, jnp.int32), bi)
            val = plsc.load_gather(hv_v, [lane, bi])
            plsc.addupdate_scatter(counts_v, [lane, bi], fone)
            return values_acc + val

        values = lax.fori_loop(0, _A, agent_body, fzero)

        mc = fzero
        for h in range(_H):
            mc = jnp.maximum(
                mc, plsc.load_gather(counts_v,
                                     [lane, jnp.full((_L,), h, jnp.int32)]))
        mf = mc * (1.0 / _A)

        e = jnp.exp(_K * (mf - _T))
        vote_part = -(values * (1.0 + e)) * (1.0 / _D)

        def mov_body(j, acc):
            jv = jnp.full((_L,), j, jnp.int32)
            mx = plsc.load_gather(mov_v, [lane, jv, izero])
            my = plsc.load_gather(mov_v, [lane, jv, ione])
            s = mx * mx + my * my
            i = plsc.bitcast(s, jnp.int32)
            i = 0x5F3759DF - lax.shift_right_logical(i, 1)
            y = plsc.bitcast(i, jnp.float32)
            for _ in range(4):
                y = y * (1.5 - 0.5 * s * y * y)
            return acc + jnp.where(s > 0.0, s * y, fzero)

        mov_part = lax.fori_loop(0, _E, mov_body, fzero)

        part_st[...] = vote_part + mov_part
        mf_st[...] = mf
        pltpu.sync_copy(part_st, part_out.at[wid])
        pltpu.sync_copy(mf_st, mf_out.at[pl.ds(base, _L)])

    return k(movements, votes, hv)


def kernel(movements, utterances, votes, hive_values, locations):
    del utterances, locations
    hv = jnp.squeeze(hive_values, axis=-1)
    partials, max_freq = _sc_bee_game(movements, votes, hv)
    return jnp.sum(partials), max_freq
